# Initial kernel scaffold; baseline (speedup 1.0000x reference)
import jax
import jax.numpy as jnp
from jax.experimental import pallas as pl


def kernel(x, edge_index, aw0, ab0, W0, b0, aw1, ab1, W1, b1):
    raise NotImplementedError("write your pallas kernel here")


# speedup vs baseline: 8.8963x; 8.8963x over previous
import functools

import jax
import jax.numpy as jnp
from jax import lax
from jax.experimental import pallas as pl
from jax.experimental.pallas import tpu as pltpu
from jax.experimental.pallas import tpu_sc as plsc

N = 10000
D = 128
NW = 32
CHUNK = 128
CPW = 80
EP = NW * CPW * CHUNK
JUNK = N
ACC_ROWS = 10240
RPS = ACC_ROWS // 16

_MESH = plsc.VectorSubcoreMesh(core_axis_name="c", subcore_axis_name="s")

def _zero_rows(ref, nrows, ncols):
  z16 = jnp.zeros((16,), jnp.float32)
  def body(i, _):
    for g in range(ncols // 16):
      ref[i, pl.ds(g * 16, 16)] = z16
    return 0
  lax.fori_loop(0, nrows, body, 0)


CACC = 2 * ACC_ROWS
CPS = CACC // 16


@functools.partial(
    pl.kernel,
    mesh=_MESH,
    out_type=jax.ShapeDtypeStruct((2 * CACC,), jnp.float32),
    scratch_types=[
        pltpu.VMEM((CHUNK,), jnp.int32),
        pltpu.VMEM((CHUNK,), jnp.int32),
        pltpu.VMEM((CHUNK,), jnp.float32),
        pltpu.VMEM((CPS,), jnp.float32),
        pltpu.VMEM_SHARED((CACC,), jnp.float32),
    ],
)
def _count_sc(rows_hbm, rows_self_hbm, out_hbm, ridx, ridx2, vbuf, zbuf, acc):
  c = lax.axis_index("c")
  s = lax.axis_index("s")
  wid = c * 16 + s
  base = s * CPS
  z16 = jnp.zeros((16,), jnp.float32)

  def zb(i, _):
    zbuf[pl.ds(i * 16, 16)] = z16
    return 0

  lax.fori_loop(0, CPS // 16, zb, 0)
  pltpu.sync_copy(zbuf, acc.at[pl.ds(base, CPS)])
  ones16 = jnp.full((16,), 1.0, jnp.float32)
  for g in range(CHUNK // 16):
    vbuf[pl.ds(g * 16, 16)] = ones16
  plsc.subcore_barrier()

  def body(j, _):
    pltpu.sync_copy(rows_hbm.at[wid, j], ridx)
    pltpu.sync_copy(rows_self_hbm.at[wid, j], ridx2)
    pltpu.sync_copy(vbuf, acc.at[ridx], add=True)
    pltpu.sync_copy(vbuf, acc.at[ridx2], add=True)
    return 0

  lax.fori_loop(0, CPW, body, 0)
  plsc.subcore_barrier()
  pltpu.sync_copy(acc.at[pl.ds(base, CPS)], zbuf)
  pltpu.sync_copy(zbuf, out_hbm.at[pl.ds(c * CACC + base, CPS)])


@functools.partial(
    pl.kernel,
    mesh=_MESH,
    out_type=jax.ShapeDtypeStruct((2, ACC_ROWS, D), jnp.float32),
    scratch_types=[
        pltpu.VMEM((CHUNK,), jnp.int32),
        pltpu.VMEM((CHUNK,), jnp.int32),
        pltpu.VMEM((CHUNK, D), jnp.float32),
        pltpu.VMEM_SHARED((ACC_ROWS, D), jnp.float32),
        pltpu.SemaphoreType.DMA,
    ],
)
def _spmm_sc(cols_hbm, rows_hbm, hp_hbm, out_hbm, cidx, ridx, gbuf, acc, sem):
  c = lax.axis_index("c")
  s = lax.axis_index("s")
  wid = c * 16 + s
  base = s * RPS
  _zero_rows(gbuf, CHUNK, D)
  for k in range(RPS // CHUNK):
    pltpu.sync_copy(gbuf, acc.at[pl.ds(base + k * CHUNK, CHUNK)])
  plsc.subcore_barrier()

  def body(j, _):
    pltpu.sync_copy(cols_hbm.at[wid, j], cidx)
    gcp = pltpu.async_copy(hp_hbm.at[cidx], gbuf, sem)
    pltpu.sync_copy(rows_hbm.at[wid, j], ridx)
    gcp.wait()
    pltpu.sync_copy(gbuf, acc.at[ridx], add=True)
    return 0

  lax.fori_loop(0, CPW, body, 0)
  plsc.subcore_barrier()
  for k in range(RPS // CHUNK):
    pltpu.sync_copy(acc.at[pl.ds(base + k * CHUNK, CHUNK)], gbuf)
    pltpu.sync_copy(gbuf, out_hbm.at[c, pl.ds(base + k * CHUNK, CHUNK)])


ROWS_BLK = 2000


def _prep_body(x_ref, cr_ref, hp_ref):
  dis = lax.rsqrt(cr_ref[...] + 1.0)
  hp_ref[...] = dis * x_ref[...]


def _prep_tc(x, cnt_row):
  return pl.pallas_call(
      _prep_body,
      grid=(N // ROWS_BLK,),
      in_specs=[
          pl.BlockSpec((ROWS_BLK, D), lambda i: (i, 0)),
          pl.BlockSpec((ROWS_BLK, 1), lambda i: (i, 0)),
      ],
      out_specs=pl.BlockSpec((ROWS_BLK, D), lambda i: (i, 0)),
      out_shape=jax.ShapeDtypeStruct((N, D), jnp.float32),
  )(x, cnt_row)


def _layer_body(h_ref, p_ref, cr_ref, cs_ref, aw_ref, ab_ref, w_ref, b_ref,
                o_ref, hp_ref, *, act):
  h = h_ref[...]
  p = p_ref[...]
  g = p[0] + p[1]
  deg = cr_ref[...] + 1.0
  dinv = 1.0 / deg
  dis = lax.rsqrt(deg)
  sc = cs_ref[...] + 1.0
  logit = jnp.sum(h * aw_ref[...], axis=1, keepdims=True) + ab_ref[0]
  alpha = jax.nn.sigmoid(logit)
  z_lp = dis * g + dinv * h
  z_mix = (2.0 * alpha - 1.0) * z_lp + (1.0 - alpha) * (sc * h)
  o = lax.dot_general(z_mix, w_ref[...], (((1,), (1,)), ((), ()))) + b_ref[...]
  if act:
    o = jnp.maximum(o, 0.0)
  o_ref[...] = o
  hp_ref[...] = dis * o


def _layer_tc(h, partials, cnt_row, cnt_self, aw, ab, w, b, act):
  dout = w.shape[0]
  body = functools.partial(_layer_body, act=act)
  return pl.pallas_call(
      body,
      grid=(N // ROWS_BLK,),
      in_specs=[
          pl.BlockSpec((ROWS_BLK, D), lambda i: (i, 0)),
          pl.BlockSpec((2, ROWS_BLK, D), lambda i: (0, i, 0)),
          pl.BlockSpec((ROWS_BLK, 1), lambda i: (i, 0)),
          pl.BlockSpec((ROWS_BLK, 1), lambda i: (i, 0)),
          pl.BlockSpec((1, D), lambda i: (0, 0)),
          pl.BlockSpec(memory_space=pltpu.SMEM),
          pl.BlockSpec((dout, D), lambda i: (0, 0)),
          pl.BlockSpec((1, dout), lambda i: (0, 0)),
      ],
      out_specs=[
          pl.BlockSpec((ROWS_BLK, dout), lambda i: (i, 0)),
          pl.BlockSpec((ROWS_BLK, dout), lambda i: (i, 0)),
      ],
      out_shape=[
          jax.ShapeDtypeStruct((N, dout), jnp.float32),
          jax.ShapeDtypeStruct((N, dout), jnp.float32),
      ],
  )(h, partials, cnt_row, cnt_self, aw, ab, w, b)


def kernel(x, edge_index, aw0, ab0, W0, b0, aw1, ab1, W1, b1):
  ei = edge_index.astype(jnp.int32)
  rows, cols = ei[0], ei[1]
  rows_self = jnp.where(rows == cols, rows + ACC_ROWS, JUNK)
  pad = EP - rows.shape[0]
  rows3 = jnp.concatenate(
      [rows, jnp.full((pad,), JUNK, jnp.int32)]).reshape(NW, CPW, CHUNK)
  cols3 = jnp.concatenate(
      [cols, jnp.zeros((pad,), jnp.int32)]).reshape(NW, CPW, CHUNK)
  rs3 = jnp.concatenate(
      [rows_self, jnp.full((pad,), JUNK, jnp.int32)]).reshape(NW, CPW, CHUNK)

  craw = _count_sc(rows3, rs3)
  cnt = craw[:CACC] + craw[CACC:]
  cnt_row = cnt[:N, None]
  cnt_self = cnt[ACC_ROWS:ACC_ROWS + N, None]

  b0_2 = b0.reshape(1, -1)
  b1_2 = b1.reshape(1, -1)

  hp0 = _prep_tc(x, cnt_row)
  part0 = _spmm_sc(cols3, rows3, hp0)
  h1, hp1 = _layer_tc(x, part0, cnt_row, cnt_self, aw0, ab0, W0, b0_2, True)
  part1 = _spmm_sc(cols3, rows3, hp1)
  h2, _ = _layer_tc(h1, part1, cnt_row, cnt_self, aw1, ab1, W1, b1_2, False)
  return h2


# speedup vs baseline: 10.1632x; 1.1424x over previous
import functools

import jax
import jax.numpy as jnp
from jax import lax
from jax.experimental import pallas as pl
from jax.experimental.pallas import tpu as pltpu
from jax.experimental.pallas import tpu_sc as plsc

N = 10000
D = 128
NW = 32
CHUNK = 128
CPW = 80
EP = NW * CPW * CHUNK
JUNK = N
ACC_ROWS = 10240
RPS = ACC_ROWS // 16

_MESH = plsc.VectorSubcoreMesh(core_axis_name="c", subcore_axis_name="s")

def _zero_rows(ref, nrows, ncols):
  z16 = jnp.zeros((16,), jnp.float32)
  def body(i, _):
    for g in range(ncols // 16):
      ref[i, pl.ds(g * 16, 16)] = z16
    return 0
  lax.fori_loop(0, nrows, body, 0)


CACC = 2 * ACC_ROWS
CPS = CACC // 16


@functools.partial(
    pl.kernel,
    mesh=_MESH,
    out_type=jax.ShapeDtypeStruct((2 * CACC,), jnp.float32),
    scratch_types=[
        pltpu.VMEM((2, CHUNK), jnp.int32),
        pltpu.VMEM((2, CHUNK), jnp.int32),
        pltpu.VMEM((CHUNK,), jnp.float32),
        pltpu.VMEM((CPS,), jnp.float32),
        pltpu.VMEM_SHARED((CACC,), jnp.float32),
        pltpu.SemaphoreType.DMA((2,)),
    ],
)
def _count_sc(rows_hbm, rows_self_hbm, out_hbm, ridxs, ridx2s, vbuf, zbuf,
              acc, sem):
  c = lax.axis_index("c")
  s = lax.axis_index("s")
  wid = c * 16 + s
  base = s * CPS
  z16 = jnp.zeros((16,), jnp.float32)

  def zb(i, _):
    zbuf[pl.ds(i * 16, 16)] = z16
    return 0

  lax.fori_loop(0, CPS // 16, zb, 0)
  pltpu.sync_copy(zbuf, acc.at[pl.ds(base, CPS)])
  ones16 = jnp.full((16,), 1.0, jnp.float32)
  for g in range(CHUNK // 16):
    vbuf[pl.ds(g * 16, 16)] = ones16
  plsc.subcore_barrier()

  def body(j, _):
    slot = lax.rem(j, 2)

    @pl.when(j >= 2)
    def _():
      pltpu.make_async_copy(vbuf, acc.at[ridxs.at[slot]], sem.at[slot]).wait()
      pltpu.make_async_copy(vbuf, acc.at[ridx2s.at[slot]],
                            sem.at[slot]).wait()

    pltpu.sync_copy(rows_hbm.at[wid, j], ridxs.at[slot])
    pltpu.sync_copy(rows_self_hbm.at[wid, j], ridx2s.at[slot])
    pltpu.async_copy(vbuf, acc.at[ridxs.at[slot]], sem.at[slot], add=True)
    pltpu.async_copy(vbuf, acc.at[ridx2s.at[slot]], sem.at[slot], add=True)
    return 0

  lax.fori_loop(0, CPW, body, 0)
  for b in range(2):
    pltpu.make_async_copy(vbuf, acc.at[ridxs.at[b]], sem.at[b]).wait()
    pltpu.make_async_copy(vbuf, acc.at[ridx2s.at[b]], sem.at[b]).wait()
  plsc.subcore_barrier()
  pltpu.sync_copy(acc.at[pl.ds(base, CPS)], zbuf)
  pltpu.sync_copy(zbuf, out_hbm.at[pl.ds(c * CACC + base, CPS)])


@functools.partial(
    pl.kernel,
    mesh=_MESH,
    out_type=jax.ShapeDtypeStruct((2, ACC_ROWS, D), jnp.float32),
    scratch_types=[
        pltpu.VMEM((2, CHUNK), jnp.int32),
        pltpu.VMEM((2, CHUNK), jnp.int32),
        pltpu.VMEM((2, CHUNK, D), jnp.float32),
        pltpu.VMEM_SHARED((ACC_ROWS, D), jnp.float32),
        pltpu.SemaphoreType.DMA((2,)),
    ],
)
def _spmm_sc(cols_hbm, rows_hbm, hp_hbm, out_hbm, cidxs, ridxs, gbufs, acc,
             semg):
  c = lax.axis_index("c")
  s = lax.axis_index("s")
  wid = c * 16 + s
  base = s * RPS
  z16 = jnp.zeros((16,), jnp.float32)

  def zb(i, _):
    for g in range(D // 16):
      gbufs[0, i, pl.ds(g * 16, 16)] = z16
    return 0

  lax.fori_loop(0, CHUNK, zb, 0)
  for k in range(RPS // CHUNK):
    pltpu.sync_copy(gbufs.at[0], acc.at[pl.ds(base + k * CHUNK, CHUNK)])
  plsc.subcore_barrier()

  pltpu.sync_copy(cols_hbm.at[wid, 0], cidxs.at[0])
  pltpu.sync_copy(rows_hbm.at[wid, 0], ridxs.at[0])
  pltpu.async_copy(hp_hbm.at[cidxs.at[0]], gbufs.at[0], semg.at[0])

  def body(j, _):
    cur = lax.rem(j, 2)
    nxt = lax.rem(j + 1, 2)

    @pl.when(j + 1 < CPW)
    def _():
      pltpu.sync_copy(cols_hbm.at[wid, j + 1], cidxs.at[nxt])
      pltpu.sync_copy(rows_hbm.at[wid, j + 1], ridxs.at[nxt])
      pltpu.async_copy(hp_hbm.at[cidxs.at[nxt]], gbufs.at[nxt], semg.at[nxt])

    pltpu.make_async_copy(hp_hbm.at[cidxs.at[cur]], gbufs.at[cur],
                          semg.at[cur]).wait()
    pltpu.sync_copy(gbufs.at[cur], acc.at[ridxs.at[cur]], add=True)
    return 0

  lax.fori_loop(0, CPW, body, 0)
  plsc.subcore_barrier()
  for k in range(RPS // CHUNK):
    pltpu.sync_copy(acc.at[pl.ds(base + k * CHUNK, CHUNK)], gbufs.at[0])
    pltpu.sync_copy(gbufs.at[0], out_hbm.at[c, pl.ds(base + k * CHUNK, CHUNK)])


ROWS_BLK = 2000


def _prep_body(x_ref, cr_ref, hp_ref):
  dis = lax.rsqrt(cr_ref[...] + 1.0)
  hp_ref[...] = dis * x_ref[...]


def _prep_tc(x, cnt_row):
  return pl.pallas_call(
      _prep_body,
      grid=(N // ROWS_BLK,),
      in_specs=[
          pl.BlockSpec((ROWS_BLK, D), lambda i: (i, 0)),
          pl.BlockSpec((ROWS_BLK, 1), lambda i: (i, 0)),
      ],
      out_specs=pl.BlockSpec((ROWS_BLK, D), lambda i: (i, 0)),
      out_shape=jax.ShapeDtypeStruct((N, D), jnp.float32),
  )(x, cnt_row)


def _layer_body(h_ref, p_ref, cr_ref, cs_ref, aw_ref, ab_ref, w_ref, b_ref,
                o_ref, hp_ref, *, act):
  h = h_ref[...]
  p = p_ref[...]
  g = p[0] + p[1]
  deg = cr_ref[...] + 1.0
  dinv = 1.0 / deg
  dis = lax.rsqrt(deg)
  sc = cs_ref[...] + 1.0
  logit = jnp.sum(h * aw_ref[...], axis=1, keepdims=True) + ab_ref[0]
  alpha = jax.nn.sigmoid(logit)
  z_lp = dis * g + dinv * h
  z_mix = (2.0 * alpha - 1.0) * z_lp + (1.0 - alpha) * (sc * h)
  o = lax.dot_general(z_mix, w_ref[...], (((1,), (1,)), ((), ()))) + b_ref[...]
  if act:
    o = jnp.maximum(o, 0.0)
  o_ref[...] = o
  hp_ref[...] = dis * o


def _layer_tc(h, partials, cnt_row, cnt_self, aw, ab, w, b, act):
  dout = w.shape[0]
  body = functools.partial(_layer_body, act=act)
  return pl.pallas_call(
      body,
      grid=(N // ROWS_BLK,),
      in_specs=[
          pl.BlockSpec((ROWS_BLK, D), lambda i: (i, 0)),
          pl.BlockSpec((2, ROWS_BLK, D), lambda i: (0, i, 0)),
          pl.BlockSpec((ROWS_BLK, 1), lambda i: (i, 0)),
          pl.BlockSpec((ROWS_BLK, 1), lambda i: (i, 0)),
          pl.BlockSpec((1, D), lambda i: (0, 0)),
          pl.BlockSpec(memory_space=pltpu.SMEM),
          pl.BlockSpec((dout, D), lambda i: (0, 0)),
          pl.BlockSpec((1, dout), lambda i: (0, 0)),
      ],
      out_specs=[
          pl.BlockSpec((ROWS_BLK, dout), lambda i: (i, 0)),
          pl.BlockSpec((ROWS_BLK, dout), lambda i: (i, 0)),
      ],
      out_shape=[
          jax.ShapeDtypeStruct((N, dout), jnp.float32),
          jax.ShapeDtypeStruct((N, dout), jnp.float32),
      ],
  )(h, partials, cnt_row, cnt_self, aw, ab, w, b)


def kernel(x, edge_index, aw0, ab0, W0, b0, aw1, ab1, W1, b1):
  ei = edge_index.astype(jnp.int32)
  rows, cols = ei[0], ei[1]
  rows_self = jnp.where(rows == cols, rows + ACC_ROWS, JUNK)
  pad = EP - rows.shape[0]
  rows3 = jnp.concatenate(
      [rows, jnp.full((pad,), JUNK, jnp.int32)]).reshape(NW, CPW, CHUNK)
  cols3 = jnp.concatenate(
      [cols, jnp.zeros((pad,), jnp.int32)]).reshape(NW, CPW, CHUNK)
  rs3 = jnp.concatenate(
      [rows_self, jnp.full((pad,), JUNK, jnp.int32)]).reshape(NW, CPW, CHUNK)

  craw = _count_sc(rows3, rs3)
  cnt = craw[:CACC] + craw[CACC:]
  cnt_row = cnt[:N, None]
  cnt_self = cnt[ACC_ROWS:ACC_ROWS + N, None]

  b0_2 = b0.reshape(1, -1)
  b1_2 = b1.reshape(1, -1)

  hp0 = _prep_tc(x, cnt_row)
  part0 = _spmm_sc(cols3, rows3, hp0)
  h1, hp1 = _layer_tc(x, part0, cnt_row, cnt_self, aw0, ab0, W0, b0_2, True)
  part1 = _spmm_sc(cols3, rows3, hp1)
  h2, _ = _layer_tc(h1, part1, cnt_row, cnt_self, aw1, ab1, W1, b1_2, False)
  return h2


# speedup vs baseline: 12.3621x; 1.2164x over previous
import functools

import jax
import jax.numpy as jnp
from jax import lax
from jax.experimental import pallas as pl
from jax.experimental.pallas import tpu as pltpu
from jax.experimental.pallas import tpu_sc as plsc

N = 10000
D = 128
NW = 32
CHUNK = 128
CPW = 80
EP = NW * CPW * CHUNK
JUNK = N
ACC_ROWS = 10240
RPS = ACC_ROWS // 16

_MESH = plsc.VectorSubcoreMesh(core_axis_name="c", subcore_axis_name="s")

def _zero_rows(ref, nrows, ncols):
  z16 = jnp.zeros((16,), jnp.float32)
  def body(i, _):
    for g in range(ncols // 16):
      ref[i, pl.ds(g * 16, 16)] = z16
    return 0
  lax.fori_loop(0, nrows, body, 0)


CACC = 2 * ACC_ROWS
CPS = CACC // 16


@functools.partial(
    pl.kernel,
    mesh=_MESH,
    out_type=jax.ShapeDtypeStruct((2 * CACC,), jnp.float32),
    scratch_types=[
        pltpu.VMEM((CPW, CHUNK), jnp.int32),
        pltpu.VMEM((CPW, CHUNK), jnp.int32),
        pltpu.VMEM((CHUNK,), jnp.float32),
        pltpu.VMEM((CPS,), jnp.float32),
        pltpu.VMEM_SHARED((CACC,), jnp.float32),
        pltpu.SemaphoreType.DMA,
    ],
)
def _count_sc(rows_hbm, rows_self_hbm, out_hbm, ridxs, ridx2s, vbuf, zbuf,
              acc, sem):
  c = lax.axis_index("c")
  s = lax.axis_index("s")
  wid = c * 16 + s
  base = s * CPS
  z16 = jnp.zeros((16,), jnp.float32)

  def zb(i, _):
    zbuf[pl.ds(i * 16, 16)] = z16
    return 0

  lax.fori_loop(0, CPS // 16, zb, 0)
  pltpu.sync_copy(zbuf, acc.at[pl.ds(base, CPS)])
  ones16 = jnp.full((16,), 1.0, jnp.float32)
  for g in range(CHUNK // 16):
    vbuf[pl.ds(g * 16, 16)] = ones16
  pltpu.sync_copy(rows_hbm.at[pl.ds(wid * CPW, CPW)], ridxs)
  pltpu.sync_copy(rows_self_hbm.at[pl.ds(wid * CPW, CPW)], ridx2s)
  plsc.subcore_barrier()

  def body(j, _):
    pltpu.async_copy(vbuf, acc.at[ridxs.at[j]], sem, add=True)
    pltpu.async_copy(vbuf, acc.at[ridx2s.at[j]], sem, add=True)

    @pl.when(j >= 4)
    def _():
      pltpu.make_async_copy(vbuf, acc.at[ridxs.at[j]], sem).wait()
      pltpu.make_async_copy(vbuf, acc.at[ridxs.at[j]], sem).wait()

    return 0

  lax.fori_loop(0, CPW, body, 0)
  for _ in range(8):
    pltpu.make_async_copy(vbuf, acc.at[ridxs.at[0]], sem).wait()
  plsc.subcore_barrier()
  pltpu.sync_copy(acc.at[pl.ds(base, CPS)], zbuf)
  pltpu.sync_copy(zbuf, out_hbm.at[pl.ds(c * CACC + base, CPS)])


CA = 120
CB = 40
NCH = 16 * (CA + CB)


K = 8


@functools.partial(
    pl.kernel,
    mesh=_MESH,
    out_type=jax.ShapeDtypeStruct((2, ACC_ROWS, D), jnp.float32),
    scratch_types=[
        pltpu.VMEM((2, K, CHUNK), jnp.int32),
        pltpu.VMEM((2, K, CHUNK), jnp.int32),
        pltpu.VMEM((2, CHUNK, D), jnp.float32),
        pltpu.VMEM_SHARED((ACC_ROWS, D), jnp.float32),
        pltpu.SemaphoreType.DMA((2,)),
        pltpu.SemaphoreType.DMA,
    ],
)
def _spmm_sc(cols_hbm, rows_hbm, hp_hbm, out_hbm, cidxb, ridxb, gbufs, acc,
             semg, semi):
  c = lax.axis_index("c")
  s = lax.axis_index("s")
  base = s * RPS
  cbase = s * (CA + CB) + c * CA
  nc = jnp.where(c == 0, CA, CB)
  z16 = jnp.zeros((16,), jnp.float32)

  def zb(i, _):
    for g in range(D // 16):
      gbufs[0, i, pl.ds(g * 16, 16)] = z16
    return 0

  lax.fori_loop(0, CHUNK, zb, 0)
  for k in range(RPS // CHUNK):
    pltpu.sync_copy(gbufs.at[0], acc.at[pl.ds(base + k * CHUNK, CHUNK)])
  plsc.subcore_barrier()

  pltpu.sync_copy(cols_hbm.at[pl.ds(cbase, K)], cidxb.at[0])
  pltpu.sync_copy(rows_hbm.at[pl.ds(cbase, K)], ridxb.at[0])
  pltpu.async_copy(cols_hbm.at[pl.ds(cbase + K, K)], cidxb.at[1], semi)
  pltpu.async_copy(rows_hbm.at[pl.ds(cbase + K, K)], ridxb.at[1], semi)

  pltpu.async_copy(hp_hbm.at[cidxb.at[0, 0]], gbufs.at[0], semg.at[0])

  def body(j, _):
    cur = lax.rem(j, 2)
    b = j // K
    off = j - b * K
    slot = lax.rem(b, 2)
    j1 = j + 1
    b1 = j1 // K
    off1 = j1 - b1 * K
    slot1 = lax.rem(b1, 2)

    @pl.when(j1 < nc)
    def _():
      @pl.when(off1 == 0)
      def _():
        pltpu.make_async_copy(cols_hbm.at[pl.ds(cbase, K)], cidxb.at[slot1],
                              semi).wait()
        pltpu.make_async_copy(rows_hbm.at[pl.ds(cbase, K)], ridxb.at[slot1],
                              semi).wait()

      @pl.when((off1 == 1) & ((b1 + 1) * K < nc))
      def _():
        nslot = lax.rem(b1 + 1, 2)
        pltpu.async_copy(cols_hbm.at[pl.ds(cbase + (b1 + 1) * K, K)],
                         cidxb.at[nslot], semi)
        pltpu.async_copy(rows_hbm.at[pl.ds(cbase + (b1 + 1) * K, K)],
                         ridxb.at[nslot], semi)

      pltpu.async_copy(hp_hbm.at[cidxb.at[slot1, off1]],
                       gbufs.at[lax.rem(j1, 2)], semg.at[lax.rem(j1, 2)])

    pltpu.make_async_copy(hp_hbm.at[cidxb.at[slot, off]], gbufs.at[cur],
                          semg.at[cur]).wait()
    pltpu.sync_copy(gbufs.at[cur], acc.at[ridxb.at[slot, off]], add=True)
    return 0

  lax.fori_loop(0, nc, body, 0)
  plsc.subcore_barrier()
  for k in range(RPS // CHUNK):
    pltpu.sync_copy(acc.at[pl.ds(base + k * CHUNK, CHUNK)], gbufs.at[0])
    pltpu.sync_copy(gbufs.at[0], out_hbm.at[c, pl.ds(base + k * CHUNK, CHUNK)])


ROWS_BLK = 2000


def _prep_body(x_ref, cr_ref, hp_ref):
  dis = lax.rsqrt(cr_ref[...] + 1.0)
  hp_ref[...] = dis * x_ref[...]


def _prep_tc(x, cnt_row):
  return pl.pallas_call(
      _prep_body,
      grid=(N // ROWS_BLK,),
      in_specs=[
          pl.BlockSpec((ROWS_BLK, D), lambda i: (i, 0)),
          pl.BlockSpec((ROWS_BLK, 1), lambda i: (i, 0)),
      ],
      out_specs=pl.BlockSpec((ROWS_BLK, D), lambda i: (i, 0)),
      out_shape=jax.ShapeDtypeStruct((N, D), jnp.float32),
  )(x, cnt_row)


def _layer_body(h_ref, p_ref, cr_ref, cs_ref, aw_ref, ab_ref, w_ref, b_ref,
                o_ref, hp_ref, *, act):
  h = h_ref[...]
  p = p_ref[...]
  g = p[0] + p[1]
  deg = cr_ref[...] + 1.0
  dinv = 1.0 / deg
  dis = lax.rsqrt(deg)
  sc = cs_ref[...] + 1.0
  logit = jnp.sum(h * aw_ref[...], axis=1, keepdims=True) + ab_ref[0]
  alpha = jax.nn.sigmoid(logit)
  z_lp = dis * g + dinv * h
  z_mix = (2.0 * alpha - 1.0) * z_lp + (1.0 - alpha) * (sc * h)
  o = lax.dot_general(z_mix, w_ref[...], (((1,), (1,)), ((), ()))) + b_ref[...]
  if act:
    o = jnp.maximum(o, 0.0)
  o_ref[...] = o
  hp_ref[...] = dis * o


def _layer_tc(h, partials, cnt_row, cnt_self, aw, ab, w, b, act):
  dout = w.shape[0]
  body = functools.partial(_layer_body, act=act)
  return pl.pallas_call(
      body,
      grid=(N // ROWS_BLK,),
      in_specs=[
          pl.BlockSpec((ROWS_BLK, D), lambda i: (i, 0)),
          pl.BlockSpec((2, ROWS_BLK, D), lambda i: (0, i, 0)),
          pl.BlockSpec((ROWS_BLK, 1), lambda i: (i, 0)),
          pl.BlockSpec((ROWS_BLK, 1), lambda i: (i, 0)),
          pl.BlockSpec((1, D), lambda i: (0, 0)),
          pl.BlockSpec(memory_space=pltpu.SMEM),
          pl.BlockSpec((dout, D), lambda i: (0, 0)),
          pl.BlockSpec((1, dout), lambda i: (0, 0)),
      ],
      out_specs=[
          pl.BlockSpec((ROWS_BLK, dout), lambda i: (i, 0)),
          pl.BlockSpec((ROWS_BLK, dout), lambda i: (i, 0)),
      ],
      out_shape=[
          jax.ShapeDtypeStruct((N, dout), jnp.float32),
          jax.ShapeDtypeStruct((N, dout), jnp.float32),
      ],
  )(h, partials, cnt_row, cnt_self, aw, ab, w, b)


def kernel(x, edge_index, aw0, ab0, W0, b0, aw1, ab1, W1, b1):
  ei = edge_index.astype(jnp.int32)
  rows, cols = ei[0], ei[1]
  rows_self = jnp.where(rows == cols, rows + ACC_ROWS, JUNK)
  pad = EP - rows.shape[0]
  rows2 = jnp.concatenate(
      [rows, jnp.full((pad,), JUNK, jnp.int32)]).reshape(NCH, CHUNK)
  cols2 = jnp.concatenate(
      [cols, jnp.zeros((pad,), jnp.int32)]).reshape(NCH, CHUNK)
  rs2 = jnp.concatenate(
      [rows_self, jnp.full((pad,), JUNK, jnp.int32)]).reshape(NCH, CHUNK)

  craw = _count_sc(rows2, rs2)
  cnt = craw[:CACC] + craw[CACC:]
  cnt_row = cnt[:N, None]
  cnt_self = cnt[ACC_ROWS:ACC_ROWS + N, None]

  b0_2 = b0.reshape(1, -1)
  b1_2 = b1.reshape(1, -1)

  hp0 = _prep_tc(x, cnt_row)
  part0 = _spmm_sc(cols2, rows2, hp0)
  h1, hp1 = _layer_tc(x, part0, cnt_row, cnt_self, aw0, ab0, W0, b0_2, True)
  part1 = _spmm_sc(cols2, rows2, hp1)
  h2, _ = _layer_tc(h1, part1, cnt_row, cnt_self, aw1, ab1, W1, b1_2, False)
  return h2


# speedup vs baseline: 14.5088x; 1.1737x over previous
import functools

import jax
import jax.numpy as jnp
from jax import lax
from jax.experimental import pallas as pl
from jax.experimental.pallas import tpu as pltpu
from jax.experimental.pallas import tpu_sc as plsc

N = 10000
D = 128
NW = 32
CHUNK = 128
CPW = 80
EP = NW * CPW * CHUNK
JUNK = N
ACC_ROWS = 10240
RPS = ACC_ROWS // 16

_MESH = plsc.VectorSubcoreMesh(core_axis_name="c", subcore_axis_name="s")

def _zero_rows(ref, nrows, ncols):
  z16 = jnp.zeros((16,), jnp.float32)
  def body(i, _):
    for g in range(ncols // 16):
      ref[i, pl.ds(g * 16, 16)] = z16
    return 0
  lax.fori_loop(0, nrows, body, 0)


CACC = 2 * ACC_ROWS
CPS = CACC // 16


@functools.partial(
    pl.kernel,
    mesh=_MESH,
    out_type=jax.ShapeDtypeStruct((2 * CACC,), jnp.float32),
    scratch_types=[
        pltpu.VMEM((CPW, CHUNK), jnp.int32),
        pltpu.VMEM((CHUNK,), jnp.float32),
        pltpu.VMEM((CPS,), jnp.float32),
        pltpu.VMEM_SHARED((CACC,), jnp.float32),
        pltpu.SemaphoreType.DMA,
    ],
)
def _count_sc(rcomb_hbm, out_hbm, ridxs, vbuf, zbuf, acc, sem):
  c = lax.axis_index("c")
  s = lax.axis_index("s")
  wid = c * 16 + s
  base = s * CPS
  z16 = jnp.zeros((16,), jnp.float32)

  def zb(i, _):
    zbuf[pl.ds(i * 16, 16)] = z16
    return 0

  lax.fori_loop(0, CPS // 16, zb, 0)
  pltpu.sync_copy(zbuf, acc.at[pl.ds(base, CPS)])
  ones16 = jnp.full((16,), 1.0, jnp.float32)
  for g in range(CHUNK // 16):
    vbuf[pl.ds(g * 16, 16)] = ones16
  pltpu.sync_copy(rcomb_hbm.at[pl.ds(wid * CPW, CPW)], ridxs)
  plsc.subcore_barrier()

  def body(j, _):
    pltpu.async_copy(vbuf, acc.at[ridxs.at[j]], sem, add=True)

    @pl.when(j >= 4)
    def _():
      pltpu.make_async_copy(vbuf, acc.at[ridxs.at[j]], sem).wait()

    return 0

  lax.fori_loop(0, CPW, body, 0)
  for _ in range(4):
    pltpu.make_async_copy(vbuf, acc.at[ridxs.at[0]], sem).wait()
  plsc.subcore_barrier()
  pltpu.sync_copy(acc.at[pl.ds(base, CPS)], zbuf)
  pltpu.sync_copy(zbuf, out_hbm.at[pl.ds(c * CACC + base, CPS)])


CA = 120
CB = 40
NCH = 16 * (CA + CB)


K = 8


@functools.partial(
    pl.kernel,
    mesh=_MESH,
    out_type=jax.ShapeDtypeStruct((2, ACC_ROWS, D), jnp.float32),
    scratch_types=[
        pltpu.VMEM((2, K, CHUNK), jnp.int32),
        pltpu.VMEM((2, K, CHUNK), jnp.int32),
        pltpu.VMEM((2, CHUNK, D), jnp.float32),
        pltpu.VMEM_SHARED((ACC_ROWS, D), jnp.float32),
        pltpu.SemaphoreType.DMA((2,)),
        pltpu.SemaphoreType.DMA((2,)),
        pltpu.SemaphoreType.DMA,
    ],
)
def _spmm_sc(cols_hbm, rows_hbm, hp_hbm, out_hbm, cidxb, ridxb, gbufs, acc,
             semg, sems, semi):
  c = lax.axis_index("c")
  s = lax.axis_index("s")
  base = s * RPS
  cbase = s * (CA + CB) + c * CA
  nc = jnp.where(c == 0, CA, CB)
  z16 = jnp.zeros((16,), jnp.float32)

  def zb(i, _):
    for g in range(D // 16):
      gbufs[0, i, pl.ds(g * 16, 16)] = z16
    return 0

  lax.fori_loop(0, CHUNK, zb, 0)
  for k in range(RPS // CHUNK):
    pltpu.sync_copy(gbufs.at[0], acc.at[pl.ds(base + k * CHUNK, CHUNK)])
  plsc.subcore_barrier()

  pltpu.sync_copy(cols_hbm.at[pl.ds(cbase, K)], cidxb.at[0])
  pltpu.sync_copy(rows_hbm.at[pl.ds(cbase, K)], ridxb.at[0])
  pltpu.async_copy(cols_hbm.at[pl.ds(cbase + K, K)], cidxb.at[1], semi)
  pltpu.async_copy(rows_hbm.at[pl.ds(cbase + K, K)], ridxb.at[1], semi)

  pltpu.async_copy(hp_hbm.at[cidxb.at[0, 0]], gbufs.at[0], semg.at[0])

  def body(j, _):
    cur = lax.rem(j, 2)
    nxt = lax.rem(j + 1, 2)
    b = j // K
    off = j - b * K
    slot = lax.rem(b, 2)
    j1 = j + 1
    b1 = j1 // K
    off1 = j1 - b1 * K
    slot1 = lax.rem(b1, 2)

    pltpu.make_async_copy(hp_hbm.at[cidxb.at[slot, off]], gbufs.at[cur],
                          semg.at[cur]).wait()
    pltpu.async_copy(gbufs.at[cur], acc.at[ridxb.at[slot, off]],
                     sems.at[cur], add=True)

    @pl.when(j1 < nc)
    def _():
      @pl.when(off1 == 0)
      def _():
        pltpu.make_async_copy(cols_hbm.at[pl.ds(cbase, K)], cidxb.at[slot1],
                              semi).wait()
        pltpu.make_async_copy(rows_hbm.at[pl.ds(cbase, K)], ridxb.at[slot1],
                              semi).wait()

      @pl.when((off1 == 1) & ((b1 + 1) * K < nc))
      def _():
        nslot = lax.rem(b1 + 1, 2)
        pltpu.async_copy(cols_hbm.at[pl.ds(cbase + (b1 + 1) * K, K)],
                         cidxb.at[nslot], semi)
        pltpu.async_copy(rows_hbm.at[pl.ds(cbase + (b1 + 1) * K, K)],
                         ridxb.at[nslot], semi)

      @pl.when(j >= 1)
      def _():
        pltpu.make_async_copy(gbufs.at[nxt], acc.at[ridxb.at[slot, off]],
                              sems.at[nxt]).wait()

      pltpu.async_copy(hp_hbm.at[cidxb.at[slot1, off1]],
                       gbufs.at[nxt], semg.at[nxt])

    return 0

  lax.fori_loop(0, nc, body, 0)
  pltpu.make_async_copy(gbufs.at[0], acc.at[ridxb.at[0, 0]], sems.at[0]).wait()
  pltpu.make_async_copy(gbufs.at[1], acc.at[ridxb.at[0, 0]], sems.at[1]).wait()
  plsc.subcore_barrier()
  for k in range(RPS // CHUNK):
    pltpu.sync_copy(acc.at[pl.ds(base + k * CHUNK, CHUNK)], gbufs.at[0])
    pltpu.sync_copy(gbufs.at[0], out_hbm.at[c, pl.ds(base + k * CHUNK, CHUNK)])


ROWS_BLK = 2000


def _prep_body(x_ref, cr_ref, hp_ref):
  dis = lax.rsqrt(cr_ref[...] + 1.0)
  hp_ref[...] = dis * x_ref[...]


def _prep_tc(x, cnt_row):
  return pl.pallas_call(
      _prep_body,
      grid=(N // ROWS_BLK,),
      in_specs=[
          pl.BlockSpec((ROWS_BLK, D), lambda i: (i, 0)),
          pl.BlockSpec((ROWS_BLK, 1), lambda i: (i, 0)),
      ],
      out_specs=pl.BlockSpec((ROWS_BLK, D), lambda i: (i, 0)),
      out_shape=jax.ShapeDtypeStruct((N, D), jnp.float32),
  )(x, cnt_row)


def _layer_body(h_ref, p_ref, cr_ref, cs_ref, aw_ref, ab_ref, w_ref, b_ref,
                o_ref, hp_ref, *, act):
  h = h_ref[...]
  p = p_ref[...]
  g = p[0] + p[1]
  deg = cr_ref[...] + 1.0
  dinv = 1.0 / deg
  dis = lax.rsqrt(deg)
  sc = cs_ref[...] + 1.0
  logit = jnp.sum(h * aw_ref[...], axis=1, keepdims=True) + ab_ref[0]
  alpha = jax.nn.sigmoid(logit)
  z_lp = dis * g + dinv * h
  z_mix = (2.0 * alpha - 1.0) * z_lp + (1.0 - alpha) * (sc * h)
  o = lax.dot_general(z_mix, w_ref[...], (((1,), (1,)), ((), ()))) + b_ref[...]
  if act:
    o = jnp.maximum(o, 0.0)
  o_ref[...] = o
  hp_ref[...] = dis * o


def _layer_tc(h, partials, cnt_row, cnt_self, aw, ab, w, b, act):
  dout = w.shape[0]
  body = functools.partial(_layer_body, act=act)
  return pl.pallas_call(
      body,
      grid=(N // ROWS_BLK,),
      in_specs=[
          pl.BlockSpec((ROWS_BLK, D), lambda i: (i, 0)),
          pl.BlockSpec((2, ROWS_BLK, D), lambda i: (0, i, 0)),
          pl.BlockSpec((ROWS_BLK, 1), lambda i: (i, 0)),
          pl.BlockSpec((ROWS_BLK, 1), lambda i: (i, 0)),
          pl.BlockSpec((1, D), lambda i: (0, 0)),
          pl.BlockSpec(memory_space=pltpu.SMEM),
          pl.BlockSpec((dout, D), lambda i: (0, 0)),
          pl.BlockSpec((1, dout), lambda i: (0, 0)),
      ],
      out_specs=[
          pl.BlockSpec((ROWS_BLK, dout), lambda i: (i, 0)),
          pl.BlockSpec((ROWS_BLK, dout), lambda i: (i, 0)),
      ],
      out_shape=[
          jax.ShapeDtypeStruct((N, dout), jnp.float32),
          jax.ShapeDtypeStruct((N, dout), jnp.float32),
      ],
  )(h, partials, cnt_row, cnt_self, aw, ab, w, b)


def kernel(x, edge_index, aw0, ab0, W0, b0, aw1, ab1, W1, b1):
  ei = edge_index.astype(jnp.int32)
  rows, cols = ei[0], ei[1]
  rcomb = jnp.where(rows == cols, rows + ACC_ROWS, rows)
  pad = EP - rows.shape[0]
  rows2 = jnp.concatenate(
      [rows, jnp.full((pad,), JUNK, jnp.int32)]).reshape(NCH, CHUNK)
  cols2 = jnp.concatenate(
      [cols, jnp.zeros((pad,), jnp.int32)]).reshape(NCH, CHUNK)
  rc2 = jnp.concatenate(
      [rcomb, jnp.full((pad,), JUNK, jnp.int32)]).reshape(NCH, CHUNK)

  craw = _count_sc(rc2)
  cnt = craw[:CACC] + craw[CACC:]
  cnt_self_col = cnt[ACC_ROWS:ACC_ROWS + N]
  cnt_row = (cnt[:N] + cnt_self_col)[:, None]
  cnt_self = cnt_self_col[:, None]

  b0_2 = b0.reshape(1, -1)
  b1_2 = b1.reshape(1, -1)

  hp0 = _prep_tc(x, cnt_row)
  part0 = _spmm_sc(cols2, rows2, hp0)
  h1, hp1 = _layer_tc(x, part0, cnt_row, cnt_self, aw0, ab0, W0, b0_2, True)
  part1 = _spmm_sc(cols2, rows2, hp1)
  h2, _ = _layer_tc(h1, part1, cnt_row, cnt_self, aw1, ab1, W1, b1_2, False)
  return h2


# speedup vs baseline: 14.8530x; 1.0237x over previous
import functools

import jax
import jax.numpy as jnp
from jax import lax
from jax.experimental import pallas as pl
from jax.experimental.pallas import tpu as pltpu
from jax.experimental.pallas import tpu_sc as plsc

N = 10000
D = 128
NW = 32
CHUNK = 128
CPW = 80
EP = NW * CPW * CHUNK
JUNK = N
ACC_ROWS = 10240
RPS = ACC_ROWS // 16

_MESH = plsc.VectorSubcoreMesh(core_axis_name="c", subcore_axis_name="s")

def _zero_rows(ref, nrows, ncols):
  z16 = jnp.zeros((16,), jnp.float32)
  def body(i, _):
    for g in range(ncols // 16):
      ref[i, pl.ds(g * 16, 16)] = z16
    return 0
  lax.fori_loop(0, nrows, body, 0)


CACC = 2 * ACC_ROWS
CPS = CACC // 16


@functools.partial(
    pl.kernel,
    mesh=_MESH,
    out_type=jax.ShapeDtypeStruct((2 * CACC,), jnp.float32),
    scratch_types=[
        pltpu.VMEM((CPW, CHUNK), jnp.int32),
        pltpu.VMEM((CHUNK,), jnp.float32),
        pltpu.VMEM((CPS,), jnp.float32),
        pltpu.VMEM_SHARED((CACC,), jnp.float32),
        pltpu.SemaphoreType.DMA,
    ],
)
def _count_sc(rcomb_hbm, out_hbm, ridxs, vbuf, zbuf, acc, sem):
  c = lax.axis_index("c")
  s = lax.axis_index("s")
  wid = c * 16 + s
  base = s * CPS
  z16 = jnp.zeros((16,), jnp.float32)

  def zb(i, _):
    zbuf[pl.ds(i * 16, 16)] = z16
    return 0

  lax.fori_loop(0, CPS // 16, zb, 0)
  pltpu.sync_copy(zbuf, acc.at[pl.ds(base, CPS)])
  ones16 = jnp.full((16,), 1.0, jnp.float32)
  for g in range(CHUNK // 16):
    vbuf[pl.ds(g * 16, 16)] = ones16
  pltpu.sync_copy(rcomb_hbm.at[pl.ds(wid * CPW, CPW)], ridxs)
  plsc.subcore_barrier()

  def body(j, _):
    pltpu.async_copy(vbuf, acc.at[ridxs.at[j]], sem, add=True)

    @pl.when(j >= 4)
    def _():
      pltpu.make_async_copy(vbuf, acc.at[ridxs.at[j]], sem).wait()

    return 0

  lax.fori_loop(0, CPW, body, 0)
  for _ in range(4):
    pltpu.make_async_copy(vbuf, acc.at[ridxs.at[0]], sem).wait()
  plsc.subcore_barrier()
  pltpu.sync_copy(acc.at[pl.ds(base, CPS)], zbuf)
  pltpu.sync_copy(zbuf, out_hbm.at[pl.ds(c * CACC + base, CPS)])


GCH = 64
CA = 240
CB = 80
NCH = 16 * (CA + CB)


K = 16
NB = 4


@functools.partial(
    pl.kernel,
    mesh=_MESH,
    out_type=jax.ShapeDtypeStruct((2, ACC_ROWS, D), jnp.float32),
    scratch_types=[
        pltpu.VMEM((2, K, GCH), jnp.int32),
        pltpu.VMEM((2, K, GCH), jnp.int32),
        pltpu.VMEM((NB, GCH, D), jnp.float32),
        pltpu.VMEM_SHARED((ACC_ROWS, D), jnp.float32),
        pltpu.SemaphoreType.DMA((NB,)),
        pltpu.SemaphoreType.DMA((NB,)),
        pltpu.SemaphoreType.DMA,
    ],
)
def _spmm_sc(cols_hbm, rows_hbm, hp_hbm, out_hbm, cidxb, ridxb, gbufs, acc,
             semg, sems, semi):
  c = lax.axis_index("c")
  s = lax.axis_index("s")
  base = s * RPS
  cbase = s * (CA + CB) + c * CA
  nc = jnp.where(c == 0, CA, CB)
  z16 = jnp.zeros((16,), jnp.float32)

  def zb(i, _):
    for g in range(D // 16):
      gbufs[0, i, pl.ds(g * 16, 16)] = z16
    return 0

  lax.fori_loop(0, GCH, zb, 0)
  for k in range(RPS // GCH):
    pltpu.sync_copy(gbufs.at[0], acc.at[pl.ds(base + k * GCH, GCH)])
  plsc.subcore_barrier()

  pltpu.sync_copy(cols_hbm.at[pl.ds(cbase, K)], cidxb.at[0])
  pltpu.sync_copy(rows_hbm.at[pl.ds(cbase, K)], ridxb.at[0])
  pltpu.async_copy(cols_hbm.at[pl.ds(cbase + K, K)], cidxb.at[1], semi)
  pltpu.async_copy(rows_hbm.at[pl.ds(cbase + K, K)], ridxb.at[1], semi)

  pltpu.async_copy(hp_hbm.at[cidxb.at[0, 0]], gbufs.at[0], semg.at[0])
  pltpu.async_copy(hp_hbm.at[cidxb.at[0, 1]], gbufs.at[1], semg.at[1])

  def body(j, _):
    cur = lax.rem(j, NB)
    b = j // K
    off = j - b * K
    slot = lax.rem(b, 2)
    j2 = j + 2
    b2 = j2 // K
    off2 = j2 - b2 * K
    slot2 = lax.rem(b2, 2)

    pltpu.make_async_copy(hp_hbm.at[cidxb.at[slot, off]], gbufs.at[cur],
                          semg.at[cur]).wait()
    pltpu.async_copy(gbufs.at[cur], acc.at[ridxb.at[slot, off]],
                     sems.at[cur], add=True)

    @pl.when(j2 < nc)
    def _():
      @pl.when(off2 == 0)
      def _():
        pltpu.make_async_copy(cols_hbm.at[pl.ds(cbase, K)], cidxb.at[slot2],
                              semi).wait()
        pltpu.make_async_copy(rows_hbm.at[pl.ds(cbase, K)], ridxb.at[slot2],
                              semi).wait()

      @pl.when((off2 == 4) & ((b2 + 1) * K < nc))
      def _():
        nslot = lax.rem(b2 + 1, 2)
        pltpu.async_copy(cols_hbm.at[pl.ds(cbase + (b2 + 1) * K, K)],
                         cidxb.at[nslot], semi)
        pltpu.async_copy(rows_hbm.at[pl.ds(cbase + (b2 + 1) * K, K)],
                         ridxb.at[nslot], semi)

      r2 = lax.rem(j2, NB)

      @pl.when(j >= 2)
      def _():
        pltpu.make_async_copy(gbufs.at[r2], acc.at[ridxb.at[slot, off]],
                              sems.at[r2]).wait()

      pltpu.async_copy(hp_hbm.at[cidxb.at[slot2, off2]],
                       gbufs.at[r2], semg.at[r2])

    return 0

  lax.fori_loop(0, nc, body, 0)
  for r in range(NB):
    pltpu.make_async_copy(gbufs.at[r], acc.at[ridxb.at[0, 0]],
                          sems.at[r]).wait()
  plsc.subcore_barrier()
  for k in range(RPS // GCH):
    pltpu.sync_copy(acc.at[pl.ds(base + k * GCH, GCH)], gbufs.at[0])
    pltpu.sync_copy(gbufs.at[0], out_hbm.at[c, pl.ds(base + k * GCH, GCH)])


ROWS_BLK = 2000


def _prep_body(x_ref, cr_ref, hp_ref):
  dis = lax.rsqrt(cr_ref[...] + 1.0)
  hp_ref[...] = dis * x_ref[...]


def _prep_tc(x, cnt_row):
  return pl.pallas_call(
      _prep_body,
      grid=(N // ROWS_BLK,),
      in_specs=[
          pl.BlockSpec((ROWS_BLK, D), lambda i: (i, 0)),
          pl.BlockSpec((ROWS_BLK, 1), lambda i: (i, 0)),
      ],
      out_specs=pl.BlockSpec((ROWS_BLK, D), lambda i: (i, 0)),
      out_shape=jax.ShapeDtypeStruct((N, D), jnp.float32),
  )(x, cnt_row)


def _layer_body(h_ref, p_ref, cr_ref, cs_ref, aw_ref, ab_ref, w_ref, b_ref,
                o_ref, hp_ref, *, act):
  h = h_ref[...]
  p = p_ref[...]
  g = p[0] + p[1]
  deg = cr_ref[...] + 1.0
  dinv = 1.0 / deg
  dis = lax.rsqrt(deg)
  sc = cs_ref[...] + 1.0
  logit = jnp.sum(h * aw_ref[...], axis=1, keepdims=True) + ab_ref[0]
  alpha = jax.nn.sigmoid(logit)
  z_lp = dis * g + dinv * h
  z_mix = (2.0 * alpha - 1.0) * z_lp + (1.0 - alpha) * (sc * h)
  o = lax.dot_general(z_mix, w_ref[...], (((1,), (1,)), ((), ()))) + b_ref[...]
  if act:
    o = jnp.maximum(o, 0.0)
  o_ref[...] = o
  hp_ref[...] = dis * o


def _layer_tc(h, partials, cnt_row, cnt_self, aw, ab, w, b, act):
  dout = w.shape[0]
  body = functools.partial(_layer_body, act=act)
  return pl.pallas_call(
      body,
      grid=(N // ROWS_BLK,),
      in_specs=[
          pl.BlockSpec((ROWS_BLK, D), lambda i: (i, 0)),
          pl.BlockSpec((2, ROWS_BLK, D), lambda i: (0, i, 0)),
          pl.BlockSpec((ROWS_BLK, 1), lambda i: (i, 0)),
          pl.BlockSpec((ROWS_BLK, 1), lambda i: (i, 0)),
          pl.BlockSpec((1, D), lambda i: (0, 0)),
          pl.BlockSpec(memory_space=pltpu.SMEM),
          pl.BlockSpec((dout, D), lambda i: (0, 0)),
          pl.BlockSpec((1, dout), lambda i: (0, 0)),
      ],
      out_specs=[
          pl.BlockSpec((ROWS_BLK, dout), lambda i: (i, 0)),
          pl.BlockSpec((ROWS_BLK, dout), lambda i: (i, 0)),
      ],
      out_shape=[
          jax.ShapeDtypeStruct((N, dout), jnp.float32),
          jax.ShapeDtypeStruct((N, dout), jnp.float32),
      ],
  )(h, partials, cnt_row, cnt_self, aw, ab, w, b)


def kernel(x, edge_index, aw0, ab0, W0, b0, aw1, ab1, W1, b1):
  ei = edge_index.astype(jnp.int32)
  rows, cols = ei[0], ei[1]
  rcomb = jnp.where(rows == cols, rows + ACC_ROWS, rows)
  pad = EP - rows.shape[0]
  rows2 = jnp.concatenate(
      [rows, jnp.full((pad,), JUNK, jnp.int32)]).reshape(NCH, GCH)
  cols2 = jnp.concatenate(
      [cols, jnp.zeros((pad,), jnp.int32)]).reshape(NCH, GCH)
  rc2 = jnp.concatenate(
      [rcomb, jnp.full((pad,), JUNK, jnp.int32)]).reshape(NW * CPW, CHUNK)

  craw = _count_sc(rc2)
  cnt = craw[:CACC] + craw[CACC:]
  cnt_self_col = cnt[ACC_ROWS:ACC_ROWS + N]
  cnt_row = (cnt[:N] + cnt_self_col)[:, None]
  cnt_self = cnt_self_col[:, None]

  b0_2 = b0.reshape(1, -1)
  b1_2 = b1.reshape(1, -1)

  hp0 = _prep_tc(x, cnt_row)
  part0 = _spmm_sc(cols2, rows2, hp0)
  h1, hp1 = _layer_tc(x, part0, cnt_row, cnt_self, aw0, ab0, W0, b0_2, True)
  part1 = _spmm_sc(cols2, rows2, hp1)
  h2, _ = _layer_tc(h1, part1, cnt_row, cnt_self, aw1, ab1, W1, b1_2, False)
  return h2
